# baseline (speedup 1.0000x reference)
import functools

import jax
import jax.numpy as jnp
from jax import lax
from jax.experimental import pallas as pl
from jax.experimental.pallas import tpu as pltpu
from jax.experimental.pallas import tpu_sc as plsc

_D = 4096
_B = 257
_NC = 2
_NS = 16
_NW = _NC * _NS
_RPW = 8
_CHUNK = 2
_NCH = _RPW // _CHUNK

_mesh = plsc.VectorSubcoreMesh(core_axis_name="c", subcore_axis_name="s")


@functools.partial(
    pl.kernel,
    out_type=[
        jax.ShapeDtypeStruct((1, _D), jnp.float32),
        jax.ShapeDtypeStruct((_B - 1, _D), jnp.float32),
    ],
    mesh=_mesh,
    compiler_params=pltpu.CompilerParams(needs_layout_passes=False),
    scratch_types=(
        [pltpu.VMEM((16,), jnp.int32)]
        + [pltpu.VMEM((16,), jnp.int32) for _ in range(_NCH)]
        + [pltpu.VMEM((_CHUNK, _D), jnp.float32) for _ in range(_NCH)]
        + [
            pltpu.VMEM((1,), jnp.int32),
            pltpu.VMEM((1, _D), jnp.float32),
        ]
        + [pltpu.SemaphoreType.DMA for _ in range(_NCH)]
        + [pltpu.SemaphoreType.DMA, pltpu.SemaphoreType.DMA]
    ),
)
def _sc_gather(table_hbm, idx_hbm, pc_hbm, fields_hbm, *refs):
    idx_v16 = refs[0]
    idx_s = refs[1:1 + _NCH]
    rows = refs[1 + _NCH:1 + 2 * _NCH]
    idxp_v = refs[1 + 2 * _NCH]
    row_pc = refs[2 + 2 * _NCH]
    sem_g = refs[3 + 2 * _NCH:3 + 3 * _NCH]
    sem_p = refs[3 + 3 * _NCH]
    sem_s = refs[4 + 3 * _NCH]

    wid = lax.axis_index("s") * _NC + lax.axis_index("c")
    base = wid * _RPW
    is_w0 = wid == 0
    is_wlast = wid == _NW - 1

    @pl.when(jnp.logical_not(is_wlast))
    def _():
        pltpu.sync_copy(idx_hbm.at[pl.ds(base, 16)], idx_v16)

    @pl.when(is_wlast)
    def _():
        pltpu.sync_copy(idx_hbm.at[pl.ds(_B - 9, 8)], idx_v16.at[pl.ds(0, 8)])
        pltpu.sync_copy(idx_hbm.at[pl.ds(_B - 1, 1)], idx_v16.at[pl.ds(8, 1)])

    lanes = lax.iota(jnp.int32, 16)
    gathers = []
    for k in range(_NCH):
        idx_s[k][...] = plsc.load_gather(
            idx_v16, [jnp.minimum(lanes + 1 + k * _CHUNK, 15)])
        gathers.append(pltpu.async_copy(
            table_hbm.at[idx_s[k].at[pl.ds(0, _CHUNK)]], rows[k], sem_g[k]))

    @pl.when(is_w0)
    def _():
        pltpu.sync_copy(idx_hbm.at[pl.ds(0, 1)], idxp_v)
        pltpu.async_copy(table_hbm.at[idxp_v], row_pc, sem_p)

    scatters = []
    for k in range(_NCH):
        gathers[k].wait()
        scatters.append(pltpu.async_copy(
            rows[k], fields_hbm.at[pl.ds(base + k * _CHUNK, _CHUNK)], sem_s))

    @pl.when(is_w0)
    def _():
        pltpu.make_async_copy(table_hbm.at[idxp_v], row_pc, sem_p).wait()
        pltpu.async_copy(row_pc, pc_hbm, sem_p).wait()

    for s in scatters:
        s.wait()


def kernel(hidden_state, schema_indices):
    table = hidden_state[0]
    pc_emb, field_embs = _sc_gather(table, schema_indices)
    return (pc_emb, field_embs)


# speedup vs baseline: 1.0105x; 1.0105x over previous
import functools

import jax
import jax.numpy as jnp
from jax import lax
from jax.experimental import pallas as pl
from jax.experimental.pallas import tpu as pltpu
from jax.experimental.pallas import tpu_sc as plsc

_D = 4096
_B = 257
_NC = 2
_NS = 16
_NW = _NC * _NS
_RPW = 8
_CHUNK = 4
_NCH = _RPW // _CHUNK

_mesh = plsc.VectorSubcoreMesh(core_axis_name="c", subcore_axis_name="s")


@functools.partial(
    pl.kernel,
    out_type=[
        jax.ShapeDtypeStruct((1, _D), jnp.float32),
        jax.ShapeDtypeStruct((_B - 1, _D), jnp.float32),
    ],
    mesh=_mesh,
    compiler_params=pltpu.CompilerParams(needs_layout_passes=False),
    scratch_types=(
        [pltpu.VMEM((16,), jnp.int32)]
        + [pltpu.VMEM((16,), jnp.int32) for _ in range(_NCH)]
        + [pltpu.VMEM((_CHUNK, _D), jnp.float32) for _ in range(_NCH)]
        + [
            pltpu.VMEM((1,), jnp.int32),
            pltpu.VMEM((1, _D), jnp.float32),
        ]
        + [pltpu.SemaphoreType.DMA for _ in range(_NCH)]
        + [pltpu.SemaphoreType.DMA, pltpu.SemaphoreType.DMA]
    ),
)
def _sc_gather(table_hbm, idx_hbm, pc_hbm, fields_hbm, *refs):
    idx_v16 = refs[0]
    idx_s = refs[1:1 + _NCH]
    rows = refs[1 + _NCH:1 + 2 * _NCH]
    idxp_v = refs[1 + 2 * _NCH]
    row_pc = refs[2 + 2 * _NCH]
    sem_g = refs[3 + 2 * _NCH:3 + 3 * _NCH]
    sem_p = refs[3 + 3 * _NCH]
    sem_s = refs[4 + 3 * _NCH]

    wid = lax.axis_index("s") * _NC + lax.axis_index("c")
    base = wid * _RPW
    is_w0 = wid == 0
    is_wlast = wid == _NW - 1

    lanes = lax.iota(jnp.int32, 16)
    p0 = pltpu.async_copy(idx_hbm.at[pl.ds(base, 8)],
                          idx_v16.at[pl.ds(0, 8)], sem_s)

    @pl.when(is_w0)
    def _():
        pltpu.sync_copy(idx_hbm.at[pl.ds(0, 1)], idxp_v)
        pltpu.async_copy(table_hbm.at[idxp_v], row_pc, sem_p)

    p0.wait()
    idx_s[0][...] = plsc.load_gather(idx_v16, [jnp.minimum(lanes + 1, 15)])
    gathers = [pltpu.async_copy(
        table_hbm.at[idx_s[0].at[pl.ds(0, _CHUNK)]], rows[0], sem_g[0])]

    @pl.when(jnp.logical_not(is_wlast))
    def _():
        pltpu.sync_copy(idx_hbm.at[pl.ds(base + 8, 8)], idx_v16.at[pl.ds(8, 8)])

    @pl.when(is_wlast)
    def _():
        pltpu.sync_copy(idx_hbm.at[pl.ds(_B - 1, 1)], idx_v16.at[pl.ds(8, 1)])

    for k in range(1, _NCH):
        idx_s[k][...] = plsc.load_gather(
            idx_v16, [jnp.minimum(lanes + 1 + k * _CHUNK, 15)])
        gathers.append(pltpu.async_copy(
            table_hbm.at[idx_s[k].at[pl.ds(0, _CHUNK)]], rows[k], sem_g[k]))

    scatters = []
    for k in range(_NCH):
        gathers[k].wait()
        scatters.append(pltpu.async_copy(
            rows[k], fields_hbm.at[pl.ds(base + k * _CHUNK, _CHUNK)], sem_s))

    @pl.when(is_w0)
    def _():
        pltpu.make_async_copy(table_hbm.at[idxp_v], row_pc, sem_p).wait()
        pltpu.async_copy(row_pc, pc_hbm, sem_p).wait()

    for s in scatters:
        s.wait()


def kernel(hidden_state, schema_indices):
    table = hidden_state[0]
    pc_emb, field_embs = _sc_gather(table, schema_indices)
    return (pc_emb, field_embs)
